# baseline (speedup 1.0000x reference)
import functools
import jax
import jax.numpy as jnp
from jax import lax
from jax.experimental import pallas as pl
from jax.experimental.pallas import tpu as pltpu
from jax.experimental.pallas import tpu_sc as plsc

_NUM_BITS = 6
_NW = 32
_R = 64


def _make_sc_call(b, c):
    n = c * _NUM_BITS
    rows_per_w = b // _NW
    n_chunks = rows_per_w // _R
    groups = n // (16 * 3)
    mesh = plsc.VectorSubcoreMesh(core_axis_name="c", subcore_axis_name="s")

    @functools.partial(
        pl.kernel,
        mesh=mesh,
        out_type=jax.ShapeDtypeStruct((b, n), jnp.float32),
        scratch_types=[
            pltpu.VMEM((_R * c,), jnp.int32),
            pltpu.VMEM((_R, n), jnp.float32),
        ],
        compiler_params=pltpu.CompilerParams(needs_layout_passes=False),
    )
    def sc_demap(in_hbm, out_hbm, x_v, y_v):
        wid = lax.axis_index("s") * 2 + lax.axis_index("c")
        row0 = wid * rows_per_w
        lane = lax.broadcasted_iota(jnp.int32, (16,), 0)
        rels = [(lane + q) // _NUM_BITS for q in (0, 4, 2)]
        shifts = [(_NUM_BITS - 1) - lax.rem(lane + q, _NUM_BITS) for q in (0, 4, 2)]
        deltas = (0, 2, 5)

        def chunk_body(ch, _):
            r_g = row0 + ch * _R
            pltpu.sync_copy(in_hbm.at[pl.ds(r_g * c, _R * c)], x_v)

            def row_body(r, _):
                def group_body(g, cbase):
                    for p in range(3):
                        idx = (cbase + deltas[p]) + rels[p]
                        xg = plsc.load_gather(x_v, [idx])
                        bits = lax.shift_right_logical(xg, shifts[p]) & 1
                        y_v[r, pl.ds(g * 48 + 16 * p, 16)] = bits.astype(jnp.float32)
                    return cbase + 8

                lax.fori_loop(0, groups, group_body, r * c, unroll=5)
                return 0

            lax.fori_loop(0, _R, row_body, 0)
            pltpu.sync_copy(y_v, out_hbm.at[pl.ds(r_g, _R)])
            return 0

        lax.fori_loop(0, n_chunks, chunk_body, 0)

    return sc_demap


def kernel(inputs, demap_table):
    del demap_table
    b, c = inputs.shape
    return _make_sc_call(b, c)(inputs.reshape(-1))


# speedup vs baseline: 1.6448x; 1.6448x over previous
import functools
import jax
import jax.numpy as jnp
from jax import lax
from jax.experimental import pallas as pl
from jax.experimental.pallas import tpu as pltpu
from jax.experimental.pallas import tpu_sc as plsc

_NUM_BITS = 6
_NW = 32
_R = 64


def _make_sc_call(b, c):
    n = c * _NUM_BITS
    rows_per_w = b // _NW
    n_chunks = rows_per_w // _R
    n_vecs = (c + 15) // 16
    mesh = plsc.VectorSubcoreMesh(core_axis_name="c", subcore_axis_name="s")

    @functools.partial(
        pl.kernel,
        mesh=mesh,
        out_type=jax.ShapeDtypeStruct((b, n), jnp.float32),
        scratch_types=[
            pltpu.VMEM((_R * c + 16,), jnp.int32),
            pltpu.VMEM((_R, n), jnp.float32),
        ],
        compiler_params=pltpu.CompilerParams(needs_layout_passes=False),
    )
    def sc_demap(in_hbm, out_hbm, x_v, y_v):
        wid = lax.axis_index("s") * 2 + lax.axis_index("c")
        row0 = wid * rows_per_w
        lane = lax.broadcasted_iota(jnp.int32, (16,), 0)
        lane6 = lane * _NUM_BITS
        tail = c - 16 * (n_vecs - 1)
        tail_mask = lane < tail

        def chunk_body(ch, _):
            r_g = row0 + ch * _R
            pltpu.sync_copy(in_hbm.at[pl.ds(r_g * c, _R * c)], x_v.at[pl.ds(0, _R * c)])

            def row_body(r, _):
                rvec = lane * 0 + r
                xb = r * c
                for v in range(n_vecs):
                    x = x_v[pl.ds(xb + 16 * v, 16)]
                    full = v < n_vecs - 1
                    for j in range(_NUM_BITS):
                        bits = lax.shift_right_logical(x, _NUM_BITS - 1 - j) & 1
                        kv = lane6 + (96 * v + j)
                        if full:
                            plsc.store_scatter(
                                y_v, [rvec, kv], bits.astype(jnp.float32))
                        else:
                            plsc.store_scatter(
                                y_v, [rvec, kv], bits.astype(jnp.float32),
                                mask=tail_mask)
                return 0

            lax.fori_loop(0, _R, row_body, 0)
            pltpu.sync_copy(y_v, out_hbm.at[pl.ds(r_g, _R)])
            return 0

        lax.fori_loop(0, n_chunks, chunk_body, 0)

    return sc_demap


def kernel(inputs, demap_table):
    del demap_table
    b, c = inputs.shape
    return _make_sc_call(b, c)(inputs.reshape(-1))
